# Initial kernel scaffold; baseline (speedup 1.0000x reference)
import functools
import math

import jax
import jax.numpy as jnp
from jax import lax
from jax.experimental import pallas as pl
from jax.experimental.pallas import tpu as pltpu
from jax.experimental.pallas import tpu_sc as plsc

VOCAB = 100000
EMB = 64
SCALE = math.sqrt(EMB)

B = 4096 * 200
NC, NS = 2, 16
NW = NC * NS
B_PER_W = B // NW
CHUNK = 512
N_CHUNKS = B_PER_W // CHUNK


def _scale_body(w_ref, o_ref):
    o_ref[...] = w_ref[...] * SCALE


_scale_table = pl.pallas_call(
    _scale_body,
    grid=(100,),
    in_specs=[pl.BlockSpec((VOCAB // 100, EMB), lambda i: (i, 0))],
    out_specs=pl.BlockSpec((VOCAB // 100, EMB), lambda i: (i, 0)),
    out_shape=jax.ShapeDtypeStruct((VOCAB, EMB), jnp.float32),
)

_mesh = plsc.VectorSubcoreMesh(core_axis_name="c", subcore_axis_name="s")


@functools.partial(
    pl.kernel,
    mesh=_mesh,
    out_type=jax.ShapeDtypeStruct((B, EMB), jnp.float32),
    scratch_types=[
        pltpu.VMEM((CHUNK,), jnp.int32),
        pltpu.VMEM((CHUNK, EMB), jnp.float32),
        pltpu.SemaphoreType.DMA,
    ],
)
def _gather(idx_hbm, table_hbm, out_hbm, idx_v, rows_v, sem):
    wid = lax.axis_index("s") * NC + lax.axis_index("c")
    base = wid * B_PER_W

    def step(i, carry):
        off = base + i * CHUNK
        pltpu.sync_copy(idx_hbm.at[pl.ds(off, CHUNK)], idx_v)
        pltpu.async_copy(table_hbm.at[idx_v], rows_v, sem).wait()
        pltpu.sync_copy(rows_v, out_hbm.at[pl.ds(off, CHUNK)])
        return carry

    lax.fori_loop(0, N_CHUNKS, step, 0)


def kernel(tokens, emb_weight):
    table = _scale_table(emb_weight)
    flat = tokens.reshape(-1).astype(jnp.int32)
    out = _gather(flat, table)
    return out.reshape(tokens.shape + (EMB,))


# speedup vs baseline: 3.5121x; 3.5121x over previous
import functools
import math

import jax
import jax.numpy as jnp
from jax import lax
from jax.experimental import pallas as pl
from jax.experimental.pallas import tpu as pltpu
from jax.experimental.pallas import tpu_sc as plsc

VOCAB = 100000
EMB = 64
SCALE = math.sqrt(EMB)

B = 4096 * 200
NC, NS = 2, 16
NW = NC * NS
B_PER_W = B // NW
CHUNK = 512
N_CHUNKS = B_PER_W // CHUNK


def _scale_body(w_ref, o_ref):
    o_ref[...] = w_ref[...] * SCALE


_scale_table = pl.pallas_call(
    _scale_body,
    grid=(100,),
    in_specs=[pl.BlockSpec((VOCAB // 100, EMB), lambda i: (i, 0))],
    out_specs=pl.BlockSpec((VOCAB // 100, EMB), lambda i: (i, 0)),
    out_shape=jax.ShapeDtypeStruct((VOCAB, EMB), jnp.float32),
)

_mesh = plsc.VectorSubcoreMesh(core_axis_name="c", subcore_axis_name="s")


@functools.partial(
    pl.kernel,
    mesh=_mesh,
    out_type=jax.ShapeDtypeStruct((B, EMB), jnp.float32),
    scratch_types=[
        pltpu.VMEM((CHUNK,), jnp.int32),
        pltpu.VMEM((CHUNK, EMB), jnp.float32),
        pltpu.SemaphoreType.DMA,
    ],
    compiler_params=pltpu.CompilerParams(use_tc_tiling_on_sc=False),
)
def _gather(idx_hbm, table_hbm, out_hbm, idx_v, rows_v, sem):
    wid = lax.axis_index("s") * NC + lax.axis_index("c")
    base = wid * B_PER_W

    def step(i, carry):
        off = base + i * CHUNK
        pltpu.sync_copy(idx_hbm.at[pl.ds(off, CHUNK)], idx_v)
        pltpu.async_copy(table_hbm.at[idx_v], rows_v, sem).wait()
        pltpu.sync_copy(rows_v, out_hbm.at[pl.ds(off, CHUNK)])
        return carry

    lax.fori_loop(0, N_CHUNKS, step, 0)


def kernel(tokens, emb_weight):
    table = _scale_table(emb_weight)
    flat = tokens.reshape(-1).astype(jnp.int32)
    out = _gather(flat, table)
    return out.reshape(tokens.shape + (EMB,))


# speedup vs baseline: 3.7589x; 1.0703x over previous
import functools
import math

import jax
import jax.numpy as jnp
from jax import lax
from jax.experimental import pallas as pl
from jax.experimental.pallas import tpu as pltpu
from jax.experimental.pallas import tpu_sc as plsc

VOCAB = 100000
EMB = 64
SCALE = math.sqrt(EMB)

B = 4096 * 200
NC, NS = 2, 16
NW = NC * NS
B_PER_W = B // NW
CHUNK = 512
N_CHUNKS = B_PER_W // CHUNK


def _scale_body(w_ref, o_ref):
    o_ref[...] = w_ref[...] * SCALE


_scale_table = pl.pallas_call(
    _scale_body,
    grid=(100,),
    in_specs=[pl.BlockSpec((VOCAB // 100, EMB), lambda i: (i, 0))],
    out_specs=pl.BlockSpec((VOCAB // 100, EMB), lambda i: (i, 0)),
    out_shape=jax.ShapeDtypeStruct((VOCAB, EMB), jnp.float32),
)

_mesh = plsc.VectorSubcoreMesh(core_axis_name="c", subcore_axis_name="s")


N_PAIRS = N_CHUNKS // 2


@functools.partial(
    pl.kernel,
    mesh=_mesh,
    out_type=jax.ShapeDtypeStruct((B, EMB), jnp.float32),
    scratch_types=[
        pltpu.VMEM((B_PER_W,), jnp.int32),
        pltpu.VMEM((CHUNK, EMB), jnp.float32),
        pltpu.VMEM((CHUNK, EMB), jnp.float32),
        pltpu.SemaphoreType.DMA,
        pltpu.SemaphoreType.DMA,
    ],
    compiler_params=pltpu.CompilerParams(use_tc_tiling_on_sc=False),
)
def _gather(idx_hbm, table_hbm, out_hbm, idx_v, rows0, rows1, sem0, sem1):
    wid = lax.axis_index("s") * NC + lax.axis_index("c")
    base = wid * B_PER_W

    pltpu.sync_copy(idx_hbm.at[pl.ds(base, B_PER_W)], idx_v)

    def idx_at(c):
        return idx_v.at[pl.ds(c * CHUNK, CHUNK)]

    pltpu.async_copy(table_hbm.at[idx_at(0)], rows0, sem0)

    def pair(p, carry):
        c0 = 2 * p
        c1 = c0 + 1
        pltpu.async_copy(table_hbm.at[idx_at(c1)], rows1, sem1)
        pltpu.make_async_copy(table_hbm.at[idx_at(c0)], rows0, sem0).wait()
        pltpu.sync_copy(rows0, out_hbm.at[pl.ds(base + c0 * CHUNK, CHUNK)])

        @pl.when(p < N_PAIRS - 1)
        def _():
            pltpu.async_copy(table_hbm.at[idx_at(c0 + 2)], rows0, sem0)

        pltpu.make_async_copy(table_hbm.at[idx_at(c1)], rows1, sem1).wait()
        pltpu.sync_copy(rows1, out_hbm.at[pl.ds(base + c1 * CHUNK, CHUNK)])
        return carry

    lax.fori_loop(0, N_PAIRS, pair, 0)


def kernel(tokens, emb_weight):
    table = _scale_table(emb_weight)
    flat = tokens.reshape(-1).astype(jnp.int32)
    out = _gather(flat, table)
    return out.reshape(tokens.shape + (EMB,))
